# baseline (speedup 1.0000x reference)
import functools

import jax
import jax.numpy as jnp
from jax import lax
from jax.experimental import pallas as pl
from jax.experimental.pallas import tpu as pltpu
from jax.experimental.pallas import tpu_sc as plsc

N = 10000
NP = 10240
E = 320000
NODE_DIM = 128
EDGE_DIM = 16
H = 64
LAYERS = 4

NC = 2
NS = 16
NW = NC * NS
C = 128
NCHUNK = E // C
CP = 2560
CPW0 = 160
CPW1 = 0
CORE1_BASE = NS * CPW0
IDX_ROWS = CP + CPW0 - CPW1
EP_E = CP * C
ROWS_PER_TILE = NP // NS

_LRELU = 0.01


def _embed_body(nf, wn, bn, wm0, h_out, hw_out):
    h = jnp.dot(nf[...], wn[...], preferred_element_type=jnp.float32) + bn[...]
    h_out[...] = h
    hw_out[...] = jnp.dot(h, wm0[...], preferred_element_type=jnp.float32)


def _node_embed(node_feats, W_node, b_node, Wm0):
    blk = 2048
    grid = NP // blk
    return pl.pallas_call(
        _embed_body,
        grid=(grid,),
        in_specs=[
            pl.BlockSpec((blk, NODE_DIM), lambda i: (i, 0)),
            pl.BlockSpec((NODE_DIM, H), lambda i: (0, 0)),
            pl.BlockSpec((1, H), lambda i: (0, 0)),
            pl.BlockSpec((H, H), lambda i: (0, 0)),
        ],
        out_specs=[
            pl.BlockSpec((blk, H), lambda i: (i, 0)),
            pl.BlockSpec((blk, H), lambda i: (i, 0)),
        ],
        out_shape=[
            jax.ShapeDtypeStruct((NP, H), jnp.float32),
            jax.ShapeDtypeStruct((NP, H), jnp.float32),
        ],
    )(node_feats, W_node, b_node.reshape(1, H), Wm0)


_EBLK = 2560


def _edge_body(ef, weff, beff, o0, o1, o2, o3):
    y = jnp.dot(ef[...], weff[...], preferred_element_type=jnp.float32) + beff[...]
    m = jnp.where(pl.program_id(0) < E // _EBLK, 1.0, 0.0)
    y = y * m
    o0[...] = y[:, 0 * H:1 * H]
    o1[...] = y[:, 1 * H:2 * H]
    o2[...] = y[:, 2 * H:3 * H]
    o3[...] = y[:, 3 * H:4 * H]


def _edge_terms(edge_feats, Weff, beff):
    grid = EP_E // _EBLK
    last = E // _EBLK - 1
    return pl.pallas_call(
        _edge_body,
        grid=(grid,),
        in_specs=[
            pl.BlockSpec((_EBLK, EDGE_DIM), lambda i: (jnp.minimum(i, last), 0)),
            pl.BlockSpec((EDGE_DIM, LAYERS * H), lambda i: (0, 0)),
            pl.BlockSpec((1, LAYERS * H), lambda i: (0, 0)),
        ],
        out_specs=[pl.BlockSpec((_EBLK, H), lambda i: (i, 0))] * LAYERS,
        out_shape=[jax.ShapeDtypeStruct((EP_E, H), jnp.float32)] * LAYERS,
    )(edge_feats, Weff, beff.reshape(1, LAYERS * H))


def _update_body(h, agg, wu, bu, wm_next, h_out, hw_out):
    a = agg[0] + agg[1]
    x = jnp.dot(h[...] + a, wu[...], preferred_element_type=jnp.float32) + bu[...]
    hn = jnp.maximum(x, x * _LRELU)
    h_out[...] = hn
    hw_out[...] = jnp.dot(hn, wm_next[...], preferred_element_type=jnp.float32)


def _node_update(h, agg, Wu_l, bu_l, Wm_next):
    return pl.pallas_call(
        _update_body,
        out_shape=[
            jax.ShapeDtypeStruct((NP, H), jnp.float32),
            jax.ShapeDtypeStruct((NP, H), jnp.float32),
        ],
    )(h, agg, Wu_l, bu_l.reshape(1, H), Wm_next)


def _final_body(h, agg, wu, bu, watt, wl1, bl1, wl2, bl2, out):
    a = agg[0] + agg[1]
    x = jnp.dot(h[...] + a, wu[...], preferred_element_type=jnp.float32) + bu[...]
    hn = jnp.maximum(x, x * _LRELU)
    logits = jnp.dot(hn, watt[...], preferred_element_type=jnp.float32)
    rows = lax.broadcasted_iota(jnp.int32, (NP, 1), 0)
    logits = jnp.where(rows < N, logits, -jnp.inf)
    m = jnp.max(logits)
    p = jnp.exp(logits - m)
    attn = p / jnp.sum(p)
    sup = jnp.sum(attn * hn, axis=0, keepdims=True)
    z = jnp.dot(sup, wl1[...], preferred_element_type=jnp.float32) + bl1[...]
    z = jnp.maximum(z, 0.0)
    out[...] = jnp.dot(z, wl2[...], preferred_element_type=jnp.float32) + bl2[...]


def _final_head(h, agg, Wu_l, bu_l, w_att, Wl1, bl1, Wl2, bl2):
    return pl.pallas_call(
        _final_body,
        out_shape=jax.ShapeDtypeStruct((1, 1), jnp.float32),
    )(h, agg, Wu_l, bu_l.reshape(1, H), w_att.reshape(H, 1),
      Wl1, bl1.reshape(1, H), Wl2, bl2.reshape(1, 1))


def _sc_msg_body(hw_hbm, src_hbm, dst_hbm, ew_hbm, zeros_hbm, agg_hbm,
                 dst_v, src10, src11, src12, src13,
                 rows0, rows1, rows2, rows3, ew0, ew1, ew2, ew3,
                 dst0, dst1, dst2, dst3, agg_sh,
                 si0, si1, si2, si3, sg0, sg1, sg2, sg3,
                 se0, se1, se2, se3, ss0, ss1, ss2, ss3):
    cid = lax.axis_index("c")
    sid = lax.axis_index("s")

    srcs = [src10, src11, src12, src13]
    rows = [rows0, rows1, rows2, rows3]
    ews = [ew0, ew1, ew2, ew3]
    dsts = [dst0, dst1, dst2, dst3]
    si = [si0, si1, si2, si3]
    sg = [sg0, sg1, sg2, sg3]
    ss = [ss0, ss1, ss2, ss3]
    se = [se0, se1, se2, se3]

    pltpu.sync_copy(zeros_hbm.at[pl.ds(sid * ROWS_PER_TILE, ROWS_PER_TILE)],
                    agg_sh.at[pl.ds(sid * ROWS_PER_TILE, ROWS_PER_TILE)])
    plsc.subcore_barrier()

    def run_pipeline(n, row0):
        pltpu.sync_copy(dst_hbm.at[pl.ds(row0, n)], dst_v.at[pl.ds(0, n)])

        def issue_src(j, b):
            pltpu.async_copy(src_hbm.at[row0 + j], srcs[b], si[b])

        def wait_src(b):
            pltpu.make_async_copy(src_hbm.at[0], srcs[b], si[b]).wait()

        def issue_in(j, b):
            pltpu.async_copy(hw_hbm.at[srcs[b]], rows[b], sg[b])
            pltpu.async_copy(ew_hbm.at[pl.ds((row0 + j) * C, C)], ews[b], se[b])

        def wait_in(b):
            pltpu.make_async_copy(hw_hbm.at[srcs[b]], rows[b], sg[b]).wait()
            pltpu.make_async_copy(ew_hbm.at[pl.ds(0, C)], ews[b], se[b]).wait()

        def issue_scatter(b):
            pltpu.async_copy(rows[b], agg_sh.at[dsts[b]], ss[b], add=True)

        def wait_scatter(b):
            pltpu.make_async_copy(rows[b], agg_sh.at[dsts[b]], ss[b]).wait()

        def compute(j, b):
            r_ref = rows[b]
            e_ref = ews[b]
            d_ref = dsts[b]

            @plsc.parallel_loop(0, C // 16, step=1, unroll=2)
            def _(q):
                sl = pl.ds(q * 16, 16)
                d_ref[sl] = dst_v[j, sl]

            @plsc.parallel_loop(0, C, step=1, unroll=4)
            def _(r):
                for k in range(H // 16):
                    sl = pl.ds(k * 16, 16)
                    x = r_ref[r, sl] + e_ref[r, sl]
                    r_ref[r, sl] = jnp.maximum(x, x * _LRELU)

        issue_src(0, 0)
        issue_src(1, 1)
        issue_src(2, 2)
        wait_src(0)
        issue_in(0, 0)
        wait_src(1)
        issue_in(1, 1)
        issue_src(3, 3)
        wait_src(2)
        issue_in(2, 2)
        wait_in(0)
        compute(0, 0)
        issue_scatter(0)
        wait_src(3)
        issue_in(3, 3)
        issue_src(4, 0)
        wait_in(1)
        compute(1, 1)
        issue_scatter(1)
        wait_in(2)
        compute(2, 2)
        issue_scatter(2)

        def full_body(j, b, b1):
            wait_scatter(b1)
            wait_src(b1)
            issue_in(j + 1, b1)
            issue_src(j + 2, (b1 + 1) % 4)
            wait_in(b)
            compute(j, b)
            issue_scatter(b)

        def group(g, carry):
            for k in range(4):
                full_body(4 * g + 3 + k, (3 + k) % 4, k % 4)
            return carry
        lax.fori_loop(0, (n - 4) // 4, group, 0)

        wait_in(3)
        compute(n - 1, 3)
        issue_scatter(3)

        wait_src(0)
        wait_scatter(0)
        wait_scatter(1)
        wait_scatter(2)
        wait_scatter(3)

    @pl.when(cid == 0)
    def _():
        run_pipeline(CPW0, sid * CPW0)

    if CPW1:
        @pl.when(cid == 1)
        def _():
            run_pipeline(CPW1, CORE1_BASE + sid * CPW1)

    plsc.subcore_barrier()
    pltpu.sync_copy(agg_sh.at[pl.ds(sid * ROWS_PER_TILE, ROWS_PER_TILE)],
                    agg_hbm.at[cid, pl.ds(sid * ROWS_PER_TILE, ROWS_PER_TILE)])


@functools.cache
def _sc_msg_kernel():
    return pl.kernel(
        _sc_msg_body,
        out_type=jax.ShapeDtypeStruct((NC, NP, H), jnp.float32),
        mesh=plsc.VectorSubcoreMesh(core_axis_name="c", subcore_axis_name="s"),
        compiler_params=pltpu.CompilerParams(use_tc_tiling_on_sc=False),
        scratch_types=(
            [pltpu.VMEM((CPW0, C), jnp.int32)]
            + [pltpu.VMEM((C,), jnp.int32)] * 4
            + [pltpu.VMEM((C, H), jnp.float32)] * 8
            + [pltpu.VMEM((C,), jnp.int32)] * 4
            + [pltpu.VMEM_SHARED((NP, H), jnp.float32)]
            + [pltpu.SemaphoreType.DMA] * 16
        ),
    )


def _sc_msg(*args):
    return _sc_msg_kernel()(*args)


def kernel(graph, node_feats, edge_feats, W_node, b_node, W_edge, b_edge,
           Wm, bm, Wu, bu, w_att, Wl1, bl1, Wl2, bl2):
    src2d = jnp.pad(graph[0].reshape(NCHUNK, C), ((0, IDX_ROWS - NCHUNK), (0, 0)))
    dst2d = jnp.pad(graph[1].reshape(NCHUNK, C), ((0, IDX_ROWS - NCHUNK), (0, 0)),
                    constant_values=N)

    Weff = jnp.concatenate([W_edge @ Wm[l] for l in range(LAYERS)], axis=1)
    beff = jnp.concatenate([b_edge @ Wm[l] + bm[l] for l in range(LAYERS)])

    node_feats_p = jnp.pad(node_feats, ((0, NP - N), (0, 0)))
    h, hw = _node_embed(node_feats_p, W_node, b_node, Wm[0])
    ew = _edge_terms(edge_feats, Weff, beff)
    zeros = jnp.zeros((NP, H), jnp.float32)

    out = None
    for l in range(LAYERS):
        agg = _sc_msg(hw, src2d, dst2d, ew[l], zeros)
        if l + 1 < LAYERS:
            h, hw = _node_update(h, agg, Wu[l], bu[l], Wm[l + 1])
        else:
            out = _final_head(h, agg, Wu[l], bu[l], w_att, Wl1, bl1, Wl2, bl2)
    return out


# speedup vs baseline: 1.3110x; 1.3110x over previous
import functools

import jax
import jax.numpy as jnp
from jax import lax
from jax.experimental import pallas as pl
from jax.experimental.pallas import tpu as pltpu
from jax.experimental.pallas import tpu_sc as plsc

N = 10000
NP = 10240
E = 320000
NODE_DIM = 128
EDGE_DIM = 16
H = 64
LAYERS = 4

NC = 2
NS = 16
NW = NC * NS
C = 128
NCHUNK = E // C
CP = 2560
CPW0 = 80
CPW1 = 80
CORE1_BASE = NS * CPW0
IDX_ROWS = CP + CPW0 - CPW1
EP_E = CP * C
ROWS_PER_TILE = NP // NS

_LRELU = 0.01


def _embed_body(nf, wn, bn, wm0, h_out, hw_out):
    h = jnp.dot(nf[...], wn[...], preferred_element_type=jnp.float32) + bn[...]
    h_out[...] = h
    hw_out[...] = jnp.dot(h, wm0[...], preferred_element_type=jnp.float32)


def _node_embed(node_feats, W_node, b_node, Wm0):
    blk = 2048
    grid = NP // blk
    return pl.pallas_call(
        _embed_body,
        grid=(grid,),
        in_specs=[
            pl.BlockSpec((blk, NODE_DIM), lambda i: (i, 0)),
            pl.BlockSpec((NODE_DIM, H), lambda i: (0, 0)),
            pl.BlockSpec((1, H), lambda i: (0, 0)),
            pl.BlockSpec((H, H), lambda i: (0, 0)),
        ],
        out_specs=[
            pl.BlockSpec((blk, H), lambda i: (i, 0)),
            pl.BlockSpec((blk, H), lambda i: (i, 0)),
        ],
        out_shape=[
            jax.ShapeDtypeStruct((NP, H), jnp.float32),
            jax.ShapeDtypeStruct((NP, H), jnp.float32),
        ],
    )(node_feats, W_node, b_node.reshape(1, H), Wm0)


_EBLK2 = 1280


def _edge_body(ef2, w2, b2, o0, o1, o2, o3):
    y = jnp.dot(ef2[...], w2[...], preferred_element_type=jnp.float32) + b2[...]
    m = jnp.where(pl.program_id(0) < (E // 2) // _EBLK2, 1.0, 0.0)
    y = y * m
    o0[...] = y[:, 0 * 128:1 * 128]
    o1[...] = y[:, 1 * 128:2 * 128]
    o2[...] = y[:, 2 * 128:3 * 128]
    o3[...] = y[:, 3 * 128:4 * 128]


def _edge_terms(ef2, W2, b2):
    grid = (EP_E // 2) // _EBLK2
    last = (E // 2) // _EBLK2 - 1
    return pl.pallas_call(
        _edge_body,
        grid=(grid,),
        in_specs=[
            pl.BlockSpec((_EBLK2, 2 * EDGE_DIM), lambda i: (jnp.minimum(i, last), 0)),
            pl.BlockSpec((2 * EDGE_DIM, LAYERS * 128), lambda i: (0, 0)),
            pl.BlockSpec((1, LAYERS * 128), lambda i: (0, 0)),
        ],
        out_specs=[pl.BlockSpec((_EBLK2, 128), lambda i: (i, 0))] * LAYERS,
        out_shape=[jax.ShapeDtypeStruct((EP_E // 2, 128), jnp.float32)] * LAYERS,
    )(ef2, W2, b2.reshape(1, LAYERS * 128))


def _update_body(h, agg, wu, bu, wm_next, h_out, hw_out):
    a = agg[0] + agg[1]
    x = jnp.dot(h[...] + a, wu[...], preferred_element_type=jnp.float32) + bu[...]
    hn = jnp.maximum(x, x * _LRELU)
    h_out[...] = hn
    hw_out[...] = jnp.dot(hn, wm_next[...], preferred_element_type=jnp.float32)


def _node_update(h, agg, Wu_l, bu_l, Wm_next):
    return pl.pallas_call(
        _update_body,
        out_shape=[
            jax.ShapeDtypeStruct((NP, H), jnp.float32),
            jax.ShapeDtypeStruct((NP, H), jnp.float32),
        ],
    )(h, agg, Wu_l, bu_l.reshape(1, H), Wm_next)


def _final_body(h, agg, wu, bu, watt, wl1, bl1, wl2, bl2, out):
    a = agg[0] + agg[1]
    x = jnp.dot(h[...] + a, wu[...], preferred_element_type=jnp.float32) + bu[...]
    hn = jnp.maximum(x, x * _LRELU)
    logits = jnp.dot(hn, watt[...], preferred_element_type=jnp.float32)
    rows = lax.broadcasted_iota(jnp.int32, (NP, 1), 0)
    logits = jnp.where(rows < N, logits, -jnp.inf)
    m = jnp.max(logits)
    p = jnp.exp(logits - m)
    attn = p / jnp.sum(p)
    sup = jnp.sum(attn * hn, axis=0, keepdims=True)
    z = jnp.dot(sup, wl1[...], preferred_element_type=jnp.float32) + bl1[...]
    z = jnp.maximum(z, 0.0)
    out[...] = jnp.dot(z, wl2[...], preferred_element_type=jnp.float32) + bl2[...]


def _final_head(h, agg, Wu_l, bu_l, w_att, Wl1, bl1, Wl2, bl2):
    return pl.pallas_call(
        _final_body,
        out_shape=jax.ShapeDtypeStruct((1, 1), jnp.float32),
    )(h, agg, Wu_l, bu_l.reshape(1, H), w_att.reshape(H, 1),
      Wl1, bl1.reshape(1, H), Wl2, bl2.reshape(1, 1))


def _sc_msg_body(hw_hbm, src_hbm, dst_hbm, ew_hbm, zeros_hbm, agg_hbm,
                 dst_v, src10, src11, src12, src13,
                 rows0, rows1, rows2, rows3, ew0, ew1, ew2, ew3,
                 dst0, dst1, dst2, dst3, agg_sh,
                 si0, si1, si2, si3, sg0, sg1, sg2, sg3,
                 se0, se1, se2, se3, ss0, ss1, ss2, ss3):
    cid = lax.axis_index("c")
    sid = lax.axis_index("s")

    srcs = [src10, src11, src12, src13]
    rows = [rows0, rows1, rows2, rows3]
    ews = [ew0, ew1, ew2, ew3]
    dsts = [dst0, dst1, dst2, dst3]
    si = [si0, si1, si2, si3]
    sg = [sg0, sg1, sg2, sg3]
    ss = [ss0, ss1, ss2, ss3]
    se = [se0, se1, se2, se3]

    pltpu.sync_copy(zeros_hbm.at[pl.ds(sid * ROWS_PER_TILE, ROWS_PER_TILE)],
                    agg_sh.at[pl.ds(sid * ROWS_PER_TILE, ROWS_PER_TILE)])
    plsc.subcore_barrier()

    def run_pipeline(n, row0):
        pltpu.sync_copy(dst_hbm.at[pl.ds(row0, n)], dst_v.at[pl.ds(0, n)])

        def issue_src(j, b):
            pltpu.async_copy(src_hbm.at[row0 + j], srcs[b], si[b])

        def wait_src(b):
            pltpu.make_async_copy(src_hbm.at[0], srcs[b], si[b]).wait()

        def issue_in(j, b):
            pltpu.async_copy(hw_hbm.at[srcs[b]], rows[b], sg[b])
            pltpu.async_copy(ew_hbm.at[pl.ds((row0 + j) * (C // 2), C // 2)],
                             ews[b], se[b])

        def wait_in(b):
            pltpu.make_async_copy(hw_hbm.at[srcs[b]], rows[b], sg[b]).wait()
            pltpu.make_async_copy(ew_hbm.at[pl.ds(0, C // 2)], ews[b], se[b]).wait()

        def issue_scatter(b):
            pltpu.async_copy(rows[b], agg_sh.at[dsts[b]], ss[b], add=True)

        def wait_scatter(b):
            pltpu.make_async_copy(rows[b], agg_sh.at[dsts[b]], ss[b]).wait()

        def compute(j, b):
            r_ref = rows[b]
            e_ref = ews[b]
            d_ref = dsts[b]

            @plsc.parallel_loop(0, C // 16, step=1, unroll=2)
            def _(q):
                sl = pl.ds(q * 16, 16)
                d_ref[sl] = dst_v[j, sl]

            @plsc.parallel_loop(0, C // 2, step=1, unroll=2)
            def _(r2):
                for half in range(2):
                    for k in range(H // 16):
                        sl = pl.ds(k * 16, 16)
                        esl = pl.ds(half * H + k * 16, 16)
                        x = r_ref[2 * r2 + half, sl] + e_ref[r2, esl]
                        r_ref[2 * r2 + half, sl] = jnp.maximum(x, x * _LRELU)

        issue_src(0, 0)
        issue_src(1, 1)
        issue_src(2, 2)
        wait_src(0)
        issue_in(0, 0)
        wait_src(1)
        issue_in(1, 1)
        issue_src(3, 3)
        wait_src(2)
        issue_in(2, 2)
        wait_in(0)
        compute(0, 0)
        issue_scatter(0)
        wait_src(3)
        issue_in(3, 3)
        issue_src(4, 0)
        wait_in(1)
        compute(1, 1)
        issue_scatter(1)
        wait_in(2)
        compute(2, 2)
        issue_scatter(2)

        def full_body(j, b, b1):
            wait_scatter(b1)
            wait_src(b1)
            issue_in(j + 1, b1)
            issue_src(j + 2, (b1 + 1) % 4)
            wait_in(b)
            compute(j, b)
            issue_scatter(b)

        def group(g, carry):
            for k in range(4):
                full_body(4 * g + 3 + k, (3 + k) % 4, k % 4)
            return carry
        lax.fori_loop(0, (n - 4) // 4, group, 0)

        wait_in(3)
        compute(n - 1, 3)
        issue_scatter(3)

        wait_src(0)
        wait_scatter(0)
        wait_scatter(1)
        wait_scatter(2)
        wait_scatter(3)

    @pl.when(cid == 0)
    def _():
        run_pipeline(CPW0, sid * CPW0)

    if CPW1:
        @pl.when(cid == 1)
        def _():
            run_pipeline(CPW1, CORE1_BASE + sid * CPW1)

    plsc.subcore_barrier()
    pltpu.sync_copy(agg_sh.at[pl.ds(sid * ROWS_PER_TILE, ROWS_PER_TILE)],
                    agg_hbm.at[cid, pl.ds(sid * ROWS_PER_TILE, ROWS_PER_TILE)])


@functools.cache
def _sc_msg_kernel():
    return pl.kernel(
        _sc_msg_body,
        out_type=jax.ShapeDtypeStruct((NC, NP, H), jnp.float32),
        mesh=plsc.VectorSubcoreMesh(core_axis_name="c", subcore_axis_name="s"),
        compiler_params=pltpu.CompilerParams(use_tc_tiling_on_sc=False),
        scratch_types=(
            [pltpu.VMEM((CPW0, C), jnp.int32)]
            + [pltpu.VMEM((C,), jnp.int32)] * 4
            + [pltpu.VMEM((C, H), jnp.float32)] * 4
            + [pltpu.VMEM((C // 2, 2 * H), jnp.float32)] * 4
            + [pltpu.VMEM((C,), jnp.int32)] * 4
            + [pltpu.VMEM_SHARED((NP, H), jnp.float32)]
            + [pltpu.SemaphoreType.DMA] * 16
        ),
    )


def _sc_msg(*args):
    return _sc_msg_kernel()(*args)


def kernel(graph, node_feats, edge_feats, W_node, b_node, W_edge, b_edge,
           Wm, bm, Wu, bu, w_att, Wl1, bl1, Wl2, bl2):
    src2d = jnp.pad(graph[0].reshape(NCHUNK, C), ((0, IDX_ROWS - NCHUNK), (0, 0)))
    dst2d = jnp.pad(graph[1].reshape(NCHUNK, C), ((0, IDX_ROWS - NCHUNK), (0, 0)),
                    constant_values=N)

    z16 = jnp.zeros((EDGE_DIM, H), jnp.float32)
    W2s, b2s = [], []
    for l in range(LAYERS):
        weff = W_edge @ Wm[l]
        beffl = b_edge @ Wm[l] + bm[l]
        W2s.append(jnp.concatenate([
            jnp.concatenate([weff, z16], axis=1),
            jnp.concatenate([z16, weff], axis=1),
        ], axis=0))
        b2s.append(jnp.concatenate([beffl, beffl]))
    W2 = jnp.concatenate(W2s, axis=1)
    b2 = jnp.concatenate(b2s)

    node_feats_p = jnp.pad(node_feats, ((0, NP - N), (0, 0)))
    h, hw = _node_embed(node_feats_p, W_node, b_node, Wm[0])
    ef2 = edge_feats.reshape(E // 2, 2 * EDGE_DIM)
    ew = _edge_terms(ef2, W2, b2)
    zeros = jnp.zeros((NP, H), jnp.float32)

    out = None
    for l in range(LAYERS):
        agg = _sc_msg(hw, src2d, dst2d, ew[l], zeros)
        if l + 1 < LAYERS:
            h, hw = _node_update(h, agg, Wu[l], bu[l], Wm[l + 1])
        else:
            out = _final_head(h, agg, Wu[l], bu[l], w_att, Wl1, bl1, Wl2, bl2)
    return out
